# baseline (speedup 1.0000x reference)
import functools

import jax
import jax.numpy as jnp
from jax import lax
from jax.experimental import pallas as pl
from jax.experimental.pallas import tpu as pltpu
from jax.experimental.pallas import tpu_sc as plsc

_ERA = 50000
_H = 10000
_HID = 128
_NC = 2
_NS = 16
_NW = _NC * _NS
_CHUNK = 128
_EPAD = _NW * _CHUNK
_IT = False


def _sc_mesh():
    return plsc.VectorSubcoreMesh(core_axis_name="c", subcore_axis_name="s",
                                  num_cores=_NC, num_subcores=_NS)


def _mlp_ln(inputs, w1s, b1, w2, b2, *, ln=True, residuals=(), block=512,
            bf16_copy=False):
    n_rows = inputs[0].shape[0]
    dout = w2.shape[1]
    n_in = len(inputs)
    n_res = len(residuals)
    grid = ((n_rows + block - 1) // block,)

    def body(*refs):
        in_refs = refs[:n_in]
        w1_refs = refs[n_in:2 * n_in]
        b1_ref = refs[2 * n_in]
        w2_ref = refs[2 * n_in + 1]
        b2_ref = refs[2 * n_in + 2]
        res_refs = refs[2 * n_in + 3:2 * n_in + 3 + n_res]
        acc = None
        for xr, wr in zip(in_refs, w1_refs):
            xv = xr[...]
            if xv.dtype != jnp.float32:
                xv = xv.astype(jnp.float32)
            t = jnp.dot(xv, wr[...], preferred_element_type=jnp.float32)
            acc = t if acc is None else acc + t
        acc = acc + b1_ref[...]
        h = acc * jax.nn.sigmoid(acc)
        y = jnp.dot(h, w2_ref[...], preferred_element_type=jnp.float32)
        y = y + b2_ref[...]
        if ln:
            m = jnp.mean(y, axis=-1, keepdims=True)
            yc = y - m
            v = jnp.mean(yc * yc, axis=-1, keepdims=True)
            y = yc * lax.rsqrt(v + 1e-5)
        for rr in res_refs:
            y = y + rr[...]
        if bf16_copy:
            refs[-2][...] = y
            refs[-1][...] = y.astype(jnp.bfloat16)
        else:
            refs[-1][...] = y

    in_specs = [pl.BlockSpec((block, a.shape[1]), lambda i: (i, 0))
                for a in inputs]
    in_specs += [pl.BlockSpec(w.shape, lambda i: (0, 0)) for w in w1s]
    b1_2d = b1.reshape(1, -1)
    b2_2d = b2.reshape(1, -1)
    in_specs += [pl.BlockSpec(b1_2d.shape, lambda i: (0, 0)),
                 pl.BlockSpec(w2.shape, lambda i: (0, 0)),
                 pl.BlockSpec(b2_2d.shape, lambda i: (0, 0))]
    in_specs += [pl.BlockSpec((block, r.shape[1]), lambda i: (i, 0))
                 for r in residuals]
    out_spec = pl.BlockSpec((block, dout), lambda i: (i, 0))
    if bf16_copy:
        out_specs = [out_spec, out_spec]
        out_shape = [jax.ShapeDtypeStruct((n_rows, dout), jnp.float32),
                     jax.ShapeDtypeStruct((n_rows, dout), jnp.bfloat16)]
    else:
        out_specs = out_spec
        out_shape = jax.ShapeDtypeStruct((n_rows, dout), jnp.float32)
    return pl.pallas_call(
        body,
        grid=grid,
        in_specs=in_specs,
        out_specs=out_specs,
        out_shape=out_shape,
        interpret=_IT,
    )(*inputs, *w1s, b1_2d, w2, b2_2d, *residuals)


def _sc_gather2(table_a, idx_a2, table_b, idx_b2):
    nchr = idx_a2.shape[0]
    ep = nchr * _CHUNK
    d = table_a.shape[1]
    dt = table_a.dtype
    nch = nchr // _NW
    grp = 2
    ngrp, tail = divmod(nch, grp)
    mesh = _sc_mesh()

    @functools.partial(
        pl.kernel,
        out_type=(jax.ShapeDtypeStruct((ep, d), dt),
                  jax.ShapeDtypeStruct((ep, d), dt)),
        mesh=mesh,
        scratch_types=[pltpu.VMEM((grp, _CHUNK), jnp.int32),
                       pltpu.VMEM((grp, _CHUNK), jnp.int32),
                       pltpu.VMEM((grp * _CHUNK, 128), dt),
                       pltpu.VMEM((grp * _CHUNK, 128), dt),
                       pltpu.SemaphoreType.DMA,
                       pltpu.SemaphoreType.DMA],
        compiler_params=pltpu.CompilerParams(use_tc_tiling_on_sc=False),
    )
    def k(ta, ia2, tb, ib2, oa, ob, iva2, ivb2, rva, rvb, sga, sgb):
        cid = lax.axis_index("c")
        sid = lax.axis_index("s")
        wid = sid * _NC + cid
        crow0 = wid * nch

        def do_group(cr, n):
            er = cr * _CHUNK
            pltpu.sync_copy(ia2.at[pl.ds(cr, n)], iva2.at[pl.ds(0, n)])
            pltpu.sync_copy(ib2.at[pl.ds(cr, n)], ivb2.at[pl.ds(0, n)])
            for q in range(n):
                pltpu.async_copy(ta.at[iva2.at[q]],
                                 rva.at[pl.ds(q * _CHUNK, _CHUNK)], sga)
                pltpu.async_copy(tb.at[ivb2.at[q]],
                                 rvb.at[pl.ds(q * _CHUNK, _CHUNK)], sgb)
            for q in range(n):
                pltpu.make_async_copy(
                    ta.at[iva2.at[q]],
                    rva.at[pl.ds(q * _CHUNK, _CHUNK)], sga).wait()
                pltpu.make_async_copy(
                    tb.at[ivb2.at[q]],
                    rvb.at[pl.ds(q * _CHUNK, _CHUNK)], sgb).wait()
            pltpu.sync_copy(rva.at[pl.ds(0, n * _CHUNK)],
                            oa.at[pl.ds(er, n * _CHUNK)])
            pltpu.sync_copy(rvb.at[pl.ds(0, n * _CHUNK)],
                            ob.at[pl.ds(er, n * _CHUNK)])

        def body(g, carry):
            do_group(crow0 + g * grp, grp)
            return carry

        lax.fori_loop(0, ngrp, body, 0)
        if tail:
            do_group(crow0 + ngrp * grp, tail)

    return k(table_a, idx_a2, table_b, idx_b2)


def _sc_segsum(msg, dst_pad, n_dst, n_slabs):
    ep, d = msg.shape
    sw = d // n_slabs
    npad = ((n_dst + 1 + 127) // 128) * 128
    bpw = ep // _NW
    nch = bpw // _CHUNK
    rows_pt = npad // _NS
    zrows = min(1024, rows_pt)
    zeros = jnp.zeros((zrows, sw), jnp.float32)
    mesh = _sc_mesh()

    grp = 2 if sw == d else 4
    ngrp, tail = divmod(nch, grp)
    nchr = ep // _CHUNK

    @functools.partial(
        pl.kernel,
        out_type=(jax.ShapeDtypeStruct((npad, d), jnp.float32),
                  jax.ShapeDtypeStruct((npad, d), jnp.float32)),
        mesh=mesh,
        scratch_types=[pltpu.VMEM((grp, _CHUNK), jnp.int32),
                       pltpu.VMEM((grp * _CHUNK, sw), jnp.float32),
                       pltpu.SemaphoreType.DMA,
                       pltpu.VMEM_SHARED((npad, sw), jnp.float32)],
        compiler_params=pltpu.CompilerParams(use_tc_tiling_on_sc=False),
    )
    def k(mh, ih2, zh, o0, o1, iv2, rv, sa, agg):
        cid = lax.axis_index("c")
        sid = lax.axis_index("s")
        wid = sid * _NC + cid
        crow0 = wid * nch
        tile_r0 = sid * rows_pt
        for slab in range(n_slabs):
            c0 = slab * sw
            r = 0
            while r < rows_pt:
                n = min(zrows, rows_pt - r)
                pltpu.sync_copy(zh.at[pl.ds(0, n)],
                                agg.at[pl.ds(tile_r0 + r, n)])
                r += n
            plsc.subcore_barrier()

            def do_group(cr, n):
                er = cr * _CHUNK
                pltpu.sync_copy(ih2.at[pl.ds(cr, n)], iv2.at[pl.ds(0, n)])
                pltpu.sync_copy(mh.at[pl.ds(er, n * _CHUNK), pl.ds(c0, sw)],
                                rv.at[pl.ds(0, n * _CHUNK)])
                for q in range(n):
                    pltpu.async_copy(rv.at[pl.ds(q * _CHUNK, _CHUNK)],
                                     agg.at[iv2.at[q]], sa, add=True)
                for q in range(n):
                    pltpu.make_async_copy(
                        rv.at[pl.ds(q * _CHUNK, _CHUNK)],
                        agg.at[iv2.at[q]], sa).wait()

            def body(g, carry):
                do_group(crow0 + g * grp, grp)
                return carry

            lax.fori_loop(0, ngrp, body, 0)
            if tail:
                do_group(crow0 + ngrp * grp, tail)
            plsc.subcore_barrier()

            @pl.when(cid == 0)
            def _():
                pltpu.sync_copy(agg.at[pl.ds(tile_r0, rows_pt)],
                                o0.at[pl.ds(tile_r0, rows_pt), pl.ds(c0, sw)])

            @pl.when(cid == 1)
            def _():
                pltpu.sync_copy(agg.at[pl.ds(tile_r0, rows_pt)],
                                o1.at[pl.ds(tile_r0, rows_pt), pl.ds(c0, sw)])

            plsc.subcore_barrier()

    return k(msg, dst_pad, zeros)


def _w(ps, i):
    return ps[i]["W"], ps[i]["b"]


def _gnn_layer(lp, src_tab16, dst_tab16, x_dst, src0, dst0, dstN, e_parts,
               n_dst, n_slabs, extra_res=(), bf16_out=False):
    src_g, dst_g = _sc_gather2(src_tab16, src0, dst_tab16, dst0)
    w1, b1 = _w(lp["msg"], 0)
    w2, b2 = _w(lp["msg"], 1)
    ins = [src_g, dst_g] + list(e_parts)
    w1s = [w1[:_HID], w1[_HID:2 * _HID]] + [w1[2 * _HID:]] * len(e_parts)
    msg = _mlp_ln(ins, w1s, b1, w2, b2, ln=True)
    agg0, agg1 = _sc_segsum(msg, dstN, n_dst, n_slabs)
    nw1, nb1 = _w(lp["node"], 0)
    nw2, nb2 = _w(lp["node"], 1)
    out = _mlp_ln([x_dst, agg0, agg1],
                  [nw1[:_HID], nw1[_HID:], nw1[_HID:]], nb1, nw2, nb2,
                  ln=True, residuals=[x_dst] + list(extra_res),
                  bf16_copy=bf16_out)
    return out, msg


def _edge_emb(p, attr):
    w1, b1 = _w(p, 0)
    w2, b2 = _w(p, 1)
    return _mlp_ln([attr], [w1], b1, w2, b2, ln=True)


def _prep_edges(ei, attr, n_dst):
    e = ei.shape[1]
    ep = ((e + _EPAD - 1) // _EPAD) * _EPAD
    pad = ep - e
    src0 = jnp.pad(ei[0], (0, pad)).reshape(-1, _CHUNK)
    dst0 = jnp.pad(ei[1], (0, pad)).reshape(-1, _CHUNK)
    dstN = jnp.pad(ei[1], (0, pad),
                   constant_values=n_dst).reshape(-1, _CHUNK)
    attr_p = jnp.pad(attr, ((0, pad), (0, 0)))
    return src0, dst0, dstN, attr_p


def kernel(x, params, era_latlons, h_latlons, e2h_edge_attr, h2h_edge_attr,
           h2e_edge_attr, e2h_edge_index, h2h_edge_index, h2e_edge_index):
    bs = x.shape[0]
    x2 = x.reshape(bs * _ERA, -1)

    w1, b1 = _w(params["era_emb"], 0)
    w2, b2 = _w(params["era_emb"], 1)
    d_in = x2.shape[1]
    x_era, x_era16 = _mlp_ln([x2, era_latlons], [w1[:d_in], w1[d_in:]],
                             b1, w2, b2, bf16_copy=True)
    w1, b1 = _w(params["h_emb"], 0)
    w2, b2 = _w(params["h_emb"], 1)
    x_h, x_h16 = _mlp_ln([h_latlons], [w1], b1, w2, b2, bf16_copy=True)

    src0, dst0, dstN, attr_p = _prep_edges(e2h_edge_index, e2h_edge_attr, _H)
    e_fmap = _edge_emb(params["fmap"]["edge_emb"], attr_p)
    (x_latent, x_latent16), _ = _gnn_layer(
        params["fmap"]["layers"][0], x_era16, x_h16, x_h,
        src0, dst0, dstN, [e_fmap], bs * _H, 1, bf16_out=True)

    src0, dst0, dstN, attr_p = _prep_edges(h2h_edge_index, h2h_edge_attr, _H)
    e_proc = _edge_emb(params["proc"]["edge_emb"], attr_p)
    (x1, x116), msg1 = _gnn_layer(
        params["proc"]["layers"][0], x_latent16, x_latent16, x_latent,
        src0, dst0, dstN, [e_proc], bs * _H, 1, bf16_out=True)
    (x_proc, x_proc16), _ = _gnn_layer(
        params["proc"]["layers"][1], x116, x116, x1,
        src0, dst0, dstN, [e_proc, msg1], bs * _H, 1,
        extra_res=[x_latent], bf16_out=True)

    src0, dst0, dstN, attr_p = _prep_edges(h2e_edge_index, h2e_edge_attr,
                                           bs * _ERA)
    e_bmap = _edge_emb(params["bmap"]["edge_emb"], attr_p)
    x_out, _ = _gnn_layer(params["bmap"]["layers"][0], x_proc16, x_era16,
                          x_era, src0, dst0, dstN, [e_bmap], bs * _ERA, 4)

    w1, b1 = _w(params["extract"], 0)
    w2, b2 = _w(params["extract"], 1)
    out_ch = w2.shape[1]
    out = _mlp_ln([x_out], [w1], b1, w2, b2, ln=False,
                  residuals=[x2[:, :out_ch]])
    return out.reshape(bs, _ERA, out_ch)


# speedup vs baseline: 1.3003x; 1.3003x over previous
import functools

import jax
import jax.numpy as jnp
from jax import lax
from jax.experimental import pallas as pl
from jax.experimental.pallas import tpu as pltpu
from jax.experimental.pallas import tpu_sc as plsc

_ERA = 50000
_H = 10000
_HID = 128
_NC = 2
_NS = 16
_NW = _NC * _NS
_CHUNK = 128
_EPAD = _NW * _CHUNK
_IT = False


def _sc_mesh():
    return plsc.VectorSubcoreMesh(core_axis_name="c", subcore_axis_name="s",
                                  num_cores=_NC, num_subcores=_NS)


def _mlp_ln(inputs, w1s, b1, w2, b2, *, ln=True, residuals=(), block=512,
            bf16_copy=False):
    n_rows = inputs[0].shape[0]
    dout = w2.shape[1]
    n_in = len(inputs)
    n_res = len(residuals)
    grid = ((n_rows + block - 1) // block,)

    def body(*refs):
        in_refs = refs[:n_in]
        w1_refs = refs[n_in:2 * n_in]
        b1_ref = refs[2 * n_in]
        w2_ref = refs[2 * n_in + 1]
        b2_ref = refs[2 * n_in + 2]
        res_refs = refs[2 * n_in + 3:2 * n_in + 3 + n_res]
        acc = None
        for xr, wr in zip(in_refs, w1_refs):
            xv = xr[...]
            if xv.dtype != jnp.float32:
                xv = xv.astype(jnp.float32)
            t = jnp.dot(xv, wr[...], preferred_element_type=jnp.float32)
            acc = t if acc is None else acc + t
        acc = acc + b1_ref[...]
        h = acc * jax.nn.sigmoid(acc)
        y = jnp.dot(h, w2_ref[...], preferred_element_type=jnp.float32)
        y = y + b2_ref[...]
        if ln:
            m = jnp.mean(y, axis=-1, keepdims=True)
            yc = y - m
            v = jnp.mean(yc * yc, axis=-1, keepdims=True)
            y = yc * lax.rsqrt(v + 1e-5)
        for rr in res_refs:
            y = y + rr[...]
        if bf16_copy:
            refs[-2][...] = y
            refs[-1][...] = y.astype(jnp.bfloat16)
        else:
            refs[-1][...] = y

    in_specs = [pl.BlockSpec((block, a.shape[1]), lambda i: (i, 0))
                for a in inputs]
    in_specs += [pl.BlockSpec(w.shape, lambda i: (0, 0)) for w in w1s]
    b1_2d = b1.reshape(1, -1)
    b2_2d = b2.reshape(1, -1)
    in_specs += [pl.BlockSpec(b1_2d.shape, lambda i: (0, 0)),
                 pl.BlockSpec(w2.shape, lambda i: (0, 0)),
                 pl.BlockSpec(b2_2d.shape, lambda i: (0, 0))]
    in_specs += [pl.BlockSpec((block, r.shape[1]), lambda i: (i, 0))
                 for r in residuals]
    out_spec = pl.BlockSpec((block, dout), lambda i: (i, 0))
    if bf16_copy:
        out_specs = [out_spec, out_spec]
        out_shape = [jax.ShapeDtypeStruct((n_rows, dout), jnp.float32),
                     jax.ShapeDtypeStruct((n_rows, dout), jnp.bfloat16)]
    else:
        out_specs = out_spec
        out_shape = jax.ShapeDtypeStruct((n_rows, dout), jnp.float32)
    return pl.pallas_call(
        body,
        grid=grid,
        in_specs=in_specs,
        out_specs=out_specs,
        out_shape=out_shape,
        interpret=_IT,
    )(*inputs, *w1s, b1_2d, w2, b2_2d, *residuals)


def _sc_gather2(table_a, idx_a2, table_b, idx_b2):
    nchr = idx_a2.shape[0]
    ep = nchr * _CHUNK
    d = table_a.shape[1]
    dt = table_a.dtype
    nch = nchr // _NW
    grp = 3
    ngrp, tail = divmod(nch, grp)
    mesh = _sc_mesh()

    @functools.partial(
        pl.kernel,
        out_type=(jax.ShapeDtypeStruct((ep, d), dt),
                  jax.ShapeDtypeStruct((ep, d), dt)),
        mesh=mesh,
        scratch_types=[pltpu.VMEM((grp, _CHUNK), jnp.int32),
                       pltpu.VMEM((grp, _CHUNK), jnp.int32),
                       pltpu.VMEM((grp * _CHUNK, 128), dt),
                       pltpu.VMEM((grp * _CHUNK, 128), dt),
                       pltpu.SemaphoreType.DMA,
                       pltpu.SemaphoreType.DMA],
        compiler_params=pltpu.CompilerParams(use_tc_tiling_on_sc=False),
    )
    def k(ta, ia2, tb, ib2, oa, ob, iva2, ivb2, rva, rvb, sga, sgb):
        cid = lax.axis_index("c")
        sid = lax.axis_index("s")
        wid = sid * _NC + cid
        crow0 = wid * nch

        def do_group(cr, n):
            er = cr * _CHUNK
            pltpu.sync_copy(ia2.at[pl.ds(cr, n)], iva2.at[pl.ds(0, n)])
            pltpu.sync_copy(ib2.at[pl.ds(cr, n)], ivb2.at[pl.ds(0, n)])
            for q in range(n):
                pltpu.async_copy(ta.at[iva2.at[q]],
                                 rva.at[pl.ds(q * _CHUNK, _CHUNK)], sga)
                pltpu.async_copy(tb.at[ivb2.at[q]],
                                 rvb.at[pl.ds(q * _CHUNK, _CHUNK)], sgb)
            for q in range(n):
                pltpu.make_async_copy(
                    ta.at[iva2.at[q]],
                    rva.at[pl.ds(q * _CHUNK, _CHUNK)], sga).wait()
                pltpu.make_async_copy(
                    tb.at[ivb2.at[q]],
                    rvb.at[pl.ds(q * _CHUNK, _CHUNK)], sgb).wait()
            pltpu.sync_copy(rva.at[pl.ds(0, n * _CHUNK)],
                            oa.at[pl.ds(er, n * _CHUNK)])
            pltpu.sync_copy(rvb.at[pl.ds(0, n * _CHUNK)],
                            ob.at[pl.ds(er, n * _CHUNK)])

        def body(g, carry):
            do_group(crow0 + g * grp, grp)
            return carry

        lax.fori_loop(0, ngrp, body, 0)
        if tail:
            do_group(crow0 + ngrp * grp, tail)

    return k(table_a, idx_a2, table_b, idx_b2)


def _sc_segsum(msg, dst_pad, n_dst, n_slabs):
    ep, d = msg.shape
    sw = d // n_slabs
    npad = ((n_dst + 1 + 127) // 128) * 128
    bpw = ep // _NW
    nch = bpw // _CHUNK
    rows_pt = npad // _NS
    zrows = min(1024, rows_pt)
    zeros = jnp.zeros((zrows, sw), jnp.float32)
    mesh = _sc_mesh()

    grp = 2 if sw == d else 4
    ngrp, tail = divmod(nch, grp)
    nchr = ep // _CHUNK

    @functools.partial(
        pl.kernel,
        out_type=(jax.ShapeDtypeStruct((npad, d), jnp.float32),
                  jax.ShapeDtypeStruct((npad, d), jnp.float32)),
        mesh=mesh,
        scratch_types=[pltpu.VMEM((grp, _CHUNK), jnp.int32),
                       pltpu.VMEM((grp * _CHUNK, sw), jnp.float32),
                       pltpu.SemaphoreType.DMA,
                       pltpu.VMEM_SHARED((npad, sw), jnp.float32)],
        compiler_params=pltpu.CompilerParams(use_tc_tiling_on_sc=False),
    )
    def k(mh, ih2, zh, o0, o1, iv2, rv, sa, agg):
        cid = lax.axis_index("c")
        sid = lax.axis_index("s")
        wid = sid * _NC + cid
        crow0 = wid * nch
        tile_r0 = sid * rows_pt
        for slab in range(n_slabs):
            c0 = slab * sw
            r = 0
            while r < rows_pt:
                n = min(zrows, rows_pt - r)
                pltpu.sync_copy(zh.at[pl.ds(0, n)],
                                agg.at[pl.ds(tile_r0 + r, n)])
                r += n
            plsc.subcore_barrier()

            def do_group(cr, n):
                er = cr * _CHUNK
                pltpu.sync_copy(ih2.at[pl.ds(cr, n)], iv2.at[pl.ds(0, n)])
                pltpu.sync_copy(mh.at[pl.ds(er, n * _CHUNK), pl.ds(c0, sw)],
                                rv.at[pl.ds(0, n * _CHUNK)])
                for q in range(n):
                    pltpu.async_copy(rv.at[pl.ds(q * _CHUNK, _CHUNK)],
                                     agg.at[iv2.at[q]], sa, add=True)
                for q in range(n):
                    pltpu.make_async_copy(
                        rv.at[pl.ds(q * _CHUNK, _CHUNK)],
                        agg.at[iv2.at[q]], sa).wait()

            def body(g, carry):
                do_group(crow0 + g * grp, grp)
                return carry

            lax.fori_loop(0, ngrp, body, 0)
            if tail:
                do_group(crow0 + ngrp * grp, tail)
            plsc.subcore_barrier()

            @pl.when(cid == 0)
            def _():
                pltpu.sync_copy(agg.at[pl.ds(tile_r0, rows_pt)],
                                o0.at[pl.ds(tile_r0, rows_pt), pl.ds(c0, sw)])

            @pl.when(cid == 1)
            def _():
                pltpu.sync_copy(agg.at[pl.ds(tile_r0, rows_pt)],
                                o1.at[pl.ds(tile_r0, rows_pt), pl.ds(c0, sw)])

            plsc.subcore_barrier()

    return k(msg, dst_pad, zeros)


def _w(ps, i):
    return ps[i]["W"], ps[i]["b"]


def _gnn_layer(lp, src_tab16, dst_tab16, x_dst, src0, dst0, dstN, e_parts,
               n_dst, n_slabs, extra_res=(), bf16_out=False):
    src_g, dst_g = _sc_gather2(src_tab16, src0, dst_tab16, dst0)
    w1, b1 = _w(lp["msg"], 0)
    w2, b2 = _w(lp["msg"], 1)
    ins = [src_g, dst_g] + list(e_parts)
    w1s = [w1[:_HID], w1[_HID:2 * _HID]] + [w1[2 * _HID:]] * len(e_parts)
    msg = _mlp_ln(ins, w1s, b1, w2, b2, ln=True)
    agg0, agg1 = _sc_segsum(msg, dstN, n_dst, n_slabs)
    nw1, nb1 = _w(lp["node"], 0)
    nw2, nb2 = _w(lp["node"], 1)
    out = _mlp_ln([x_dst, agg0, agg1],
                  [nw1[:_HID], nw1[_HID:], nw1[_HID:]], nb1, nw2, nb2,
                  ln=True, residuals=[x_dst] + list(extra_res),
                  bf16_copy=bf16_out)
    return out, msg


def _edge_emb(p, attr):
    w1, b1 = _w(p, 0)
    w2, b2 = _w(p, 1)
    return _mlp_ln([attr], [w1], b1, w2, b2, ln=True)


def _prep_edges(ei, attr, n_dst):
    e = ei.shape[1]
    ep = ((e + _EPAD - 1) // _EPAD) * _EPAD
    pad = ep - e
    src0 = jnp.pad(ei[0], (0, pad)).reshape(-1, _CHUNK)
    dst0 = jnp.pad(ei[1], (0, pad)).reshape(-1, _CHUNK)
    dstN = jnp.pad(ei[1], (0, pad),
                   constant_values=n_dst).reshape(-1, _CHUNK)
    attr_p = jnp.pad(attr, ((0, pad), (0, 0)))
    return src0, dst0, dstN, attr_p


def kernel(x, params, era_latlons, h_latlons, e2h_edge_attr, h2h_edge_attr,
           h2e_edge_attr, e2h_edge_index, h2h_edge_index, h2e_edge_index):
    bs = x.shape[0]
    x2 = x.reshape(bs * _ERA, -1)

    w1, b1 = _w(params["era_emb"], 0)
    w2, b2 = _w(params["era_emb"], 1)
    d_in = x2.shape[1]
    x_era = _mlp_ln([x2, era_latlons], [w1[:d_in], w1[d_in:]], b1, w2, b2)
    w1, b1 = _w(params["h_emb"], 0)
    w2, b2 = _w(params["h_emb"], 1)
    x_h = _mlp_ln([h_latlons], [w1], b1, w2, b2)

    src0, dst0, dstN, attr_p = _prep_edges(e2h_edge_index, e2h_edge_attr, _H)
    e_fmap = _edge_emb(params["fmap"]["edge_emb"], attr_p)
    x_latent, _ = _gnn_layer(params["fmap"]["layers"][0], x_era, x_h, x_h,
                             src0, dst0, dstN, [e_fmap], bs * _H, 1)

    src0, dst0, dstN, attr_p = _prep_edges(h2h_edge_index, h2h_edge_attr, _H)
    e_proc = _edge_emb(params["proc"]["edge_emb"], attr_p)
    x1, msg1 = _gnn_layer(params["proc"]["layers"][0], x_latent, x_latent,
                          x_latent, src0, dst0, dstN, [e_proc], bs * _H, 1)
    x_proc, _ = _gnn_layer(params["proc"]["layers"][1], x1, x1, x1,
                           src0, dst0, dstN, [e_proc, msg1], bs * _H, 1,
                           extra_res=[x_latent])

    src0, dst0, dstN, attr_p = _prep_edges(h2e_edge_index, h2e_edge_attr,
                                           bs * _ERA)
    e_bmap = _edge_emb(params["bmap"]["edge_emb"], attr_p)
    x_out, _ = _gnn_layer(params["bmap"]["layers"][0], x_proc, x_era,
                          x_era, src0, dst0, dstN, [e_bmap], bs * _ERA, 4)

    w1, b1 = _w(params["extract"], 0)
    w2, b2 = _w(params["extract"], 1)
    out_ch = w2.shape[1]
    out = _mlp_ln([x_out], [w1], b1, w2, b2, ln=False,
                  residuals=[x2[:, :out_ch]])
    return out.reshape(bs, _ERA, out_ch)


# speedup vs baseline: 1.5924x; 1.2247x over previous
import functools

import jax
import jax.numpy as jnp
from jax import lax
from jax.experimental import pallas as pl
from jax.experimental.pallas import tpu as pltpu
from jax.experimental.pallas import tpu_sc as plsc

_ERA = 50000
_H = 10000
_HID = 128
_NC = 2
_NS = 16
_NW = _NC * _NS
_CHUNK = 128
_EPAD = _NW * _CHUNK


def _sc_mesh():
    return plsc.VectorSubcoreMesh(core_axis_name="c", subcore_axis_name="s",
                                  num_cores=_NC, num_subcores=_NS)


def _mlp_ln(inputs, w1s, b1, w2, b2, *, ln=True, residuals=(), block=1024,
            bf16_copy=False):
    n_rows = inputs[0].shape[0]
    dout = w2.shape[1]
    n_in = len(inputs)
    n_res = len(residuals)
    grid = ((n_rows + block - 1) // block,)

    def body(*refs):
        in_refs = refs[:n_in]
        w1_refs = refs[n_in:2 * n_in]
        b1_ref = refs[2 * n_in]
        w2_ref = refs[2 * n_in + 1]
        b2_ref = refs[2 * n_in + 2]
        res_refs = refs[2 * n_in + 3:2 * n_in + 3 + n_res]
        acc = None
        for xr, wr in zip(in_refs, w1_refs):
            xv = xr[...]
            if xv.dtype != jnp.float32:
                xv = xv.astype(jnp.float32)
            t = jnp.dot(xv, wr[...], preferred_element_type=jnp.float32)
            acc = t if acc is None else acc + t
        acc = acc + b1_ref[...]
        h = acc * jax.nn.sigmoid(acc)
        y = jnp.dot(h, w2_ref[...], preferred_element_type=jnp.float32)
        y = y + b2_ref[...]
        if ln:
            m = jnp.mean(y, axis=-1, keepdims=True)
            yc = y - m
            v = jnp.mean(yc * yc, axis=-1, keepdims=True)
            y = yc * lax.rsqrt(v + 1e-5)
        for rr in res_refs:
            y = y + rr[...]
        if bf16_copy:
            refs[-2][...] = y
            refs[-1][...] = y.astype(jnp.bfloat16)
        else:
            refs[-1][...] = y

    in_specs = [pl.BlockSpec((block, a.shape[1]), lambda i: (i, 0))
                for a in inputs]
    in_specs += [pl.BlockSpec(w.shape, lambda i: (0, 0)) for w in w1s]
    b1_2d = b1.reshape(1, -1)
    b2_2d = b2.reshape(1, -1)
    in_specs += [pl.BlockSpec(b1_2d.shape, lambda i: (0, 0)),
                 pl.BlockSpec(w2.shape, lambda i: (0, 0)),
                 pl.BlockSpec(b2_2d.shape, lambda i: (0, 0))]
    in_specs += [pl.BlockSpec((block, r.shape[1]), lambda i: (i, 0))
                 for r in residuals]
    out_spec = pl.BlockSpec((block, dout), lambda i: (i, 0))
    if bf16_copy:
        out_specs = [out_spec, out_spec]
        out_shape = [jax.ShapeDtypeStruct((n_rows, dout), jnp.float32),
                     jax.ShapeDtypeStruct((n_rows, dout), jnp.bfloat16)]
    else:
        out_specs = out_spec
        out_shape = jax.ShapeDtypeStruct((n_rows, dout), jnp.float32)
    return pl.pallas_call(
        body,
        grid=grid,
        in_specs=in_specs,
        out_specs=out_specs,
        out_shape=out_shape,
    )(*inputs, *w1s, b1_2d, w2, b2_2d, *residuals)


def _sc_gather2(table_a, idx_a2, table_b, idx_b2):
    nchr = idx_a2.shape[0]
    ep = nchr * _CHUNK
    d = table_a.shape[1]
    dt = table_a.dtype
    nch = nchr // _NW
    grp = 3
    ngrp, tail = divmod(nch, grp)
    mesh = _sc_mesh()

    @functools.partial(
        pl.kernel,
        out_type=(jax.ShapeDtypeStruct((ep, d), dt),
                  jax.ShapeDtypeStruct((ep, d), dt)),
        mesh=mesh,
        scratch_types=[pltpu.VMEM((grp, _CHUNK), jnp.int32),
                       pltpu.VMEM((grp, _CHUNK), jnp.int32),
                       pltpu.VMEM((grp * _CHUNK, 128), dt),
                       pltpu.VMEM((grp * _CHUNK, 128), dt),
                       pltpu.SemaphoreType.DMA,
                       pltpu.SemaphoreType.DMA],
        compiler_params=pltpu.CompilerParams(use_tc_tiling_on_sc=False),
    )
    def k(ta, ia2, tb, ib2, oa, ob, iva2, ivb2, rva, rvb, sga, sgb):
        cid = lax.axis_index("c")
        sid = lax.axis_index("s")
        wid = sid * _NC + cid
        crow0 = wid * nch

        def do_group(cr, n):
            er = cr * _CHUNK
            pltpu.sync_copy(ia2.at[pl.ds(cr, n)], iva2.at[pl.ds(0, n)])
            pltpu.sync_copy(ib2.at[pl.ds(cr, n)], ivb2.at[pl.ds(0, n)])
            for q in range(n):
                pltpu.async_copy(ta.at[iva2.at[q]],
                                 rva.at[pl.ds(q * _CHUNK, _CHUNK)], sga)
                pltpu.async_copy(tb.at[ivb2.at[q]],
                                 rvb.at[pl.ds(q * _CHUNK, _CHUNK)], sgb)
            for q in range(n):
                pltpu.make_async_copy(
                    ta.at[iva2.at[q]],
                    rva.at[pl.ds(q * _CHUNK, _CHUNK)], sga).wait()
                pltpu.make_async_copy(
                    tb.at[ivb2.at[q]],
                    rvb.at[pl.ds(q * _CHUNK, _CHUNK)], sgb).wait()
            pltpu.sync_copy(rva.at[pl.ds(0, n * _CHUNK)],
                            oa.at[pl.ds(er, n * _CHUNK)])
            pltpu.sync_copy(rvb.at[pl.ds(0, n * _CHUNK)],
                            ob.at[pl.ds(er, n * _CHUNK)])

        def body(g, carry):
            do_group(crow0 + g * grp, grp)
            return carry

        lax.fori_loop(0, ngrp, body, 0)
        if tail:
            do_group(crow0 + ngrp * grp, tail)

    return k(table_a, idx_a2, table_b, idx_b2)


def _sc_segsum(msg, dst_pad, n_dst, n_slabs):
    ep, d = msg.shape
    sw = d // n_slabs
    npad = ((n_dst + 1 + 127) // 128) * 128
    bpw = ep // _NW
    nch = bpw // _CHUNK
    rows_pt = npad // _NS
    zrows = min(1024, rows_pt)
    zeros = jnp.zeros((zrows, sw), jnp.float32)
    mesh = _sc_mesh()

    grp = 2 if sw == d else 4
    ngrp, tail = divmod(nch, grp)
    nchr = ep // _CHUNK

    @functools.partial(
        pl.kernel,
        out_type=(jax.ShapeDtypeStruct((npad, d), jnp.float32),
                  jax.ShapeDtypeStruct((npad, d), jnp.float32)),
        mesh=mesh,
        scratch_types=[pltpu.VMEM((grp, _CHUNK), jnp.int32),
                       pltpu.VMEM((grp * _CHUNK, sw), jnp.float32),
                       pltpu.SemaphoreType.DMA,
                       pltpu.VMEM_SHARED((npad, sw), jnp.float32)],
        compiler_params=pltpu.CompilerParams(use_tc_tiling_on_sc=False),
    )
    def k(mh, ih2, zh, o0, o1, iv2, rv, sa, agg):
        cid = lax.axis_index("c")
        sid = lax.axis_index("s")
        wid = sid * _NC + cid
        crow0 = wid * nch
        tile_r0 = sid * rows_pt
        for slab in range(n_slabs):
            c0 = slab * sw
            r = 0
            while r < rows_pt:
                n = min(zrows, rows_pt - r)
                pltpu.sync_copy(zh.at[pl.ds(0, n)],
                                agg.at[pl.ds(tile_r0 + r, n)])
                r += n
            plsc.subcore_barrier()

            def do_group(cr, n):
                er = cr * _CHUNK
                pltpu.sync_copy(ih2.at[pl.ds(cr, n)], iv2.at[pl.ds(0, n)])
                pltpu.sync_copy(mh.at[pl.ds(er, n * _CHUNK), pl.ds(c0, sw)],
                                rv.at[pl.ds(0, n * _CHUNK)])
                for q in range(n):
                    pltpu.async_copy(rv.at[pl.ds(q * _CHUNK, _CHUNK)],
                                     agg.at[iv2.at[q]], sa, add=True)
                for q in range(n):
                    pltpu.make_async_copy(
                        rv.at[pl.ds(q * _CHUNK, _CHUNK)],
                        agg.at[iv2.at[q]], sa).wait()

            def body(g, carry):
                do_group(crow0 + g * grp, grp)
                return carry

            lax.fori_loop(0, ngrp, body, 0)
            if tail:
                do_group(crow0 + ngrp * grp, tail)
            plsc.subcore_barrier()

            @pl.when(cid == 0)
            def _():
                pltpu.sync_copy(agg.at[pl.ds(tile_r0, rows_pt)],
                                o0.at[pl.ds(tile_r0, rows_pt), pl.ds(c0, sw)])

            @pl.when(cid == 1)
            def _():
                pltpu.sync_copy(agg.at[pl.ds(tile_r0, rows_pt)],
                                o1.at[pl.ds(tile_r0, rows_pt), pl.ds(c0, sw)])

            plsc.subcore_barrier()

    return k(msg, dst_pad, zeros)


def _w(ps, i):
    return ps[i]["W"], ps[i]["b"]


def _gnn_layer(lp, src_tab16, dst_tab16, x_dst, src0, dst0, dstN, e_parts,
               n_dst, n_slabs, extra_res=(), bf16_out=False):
    src_g, dst_g = _sc_gather2(src_tab16, src0, dst_tab16, dst0)
    w1, b1 = _w(lp["msg"], 0)
    w2, b2 = _w(lp["msg"], 1)
    ins = [src_g, dst_g] + list(e_parts)
    w1s = [w1[:_HID], w1[_HID:2 * _HID]] + [w1[2 * _HID:]] * len(e_parts)
    msg = _mlp_ln(ins, w1s, b1, w2, b2, ln=True)
    agg0, agg1 = _sc_segsum(msg, dstN, n_dst, n_slabs)
    nw1, nb1 = _w(lp["node"], 0)
    nw2, nb2 = _w(lp["node"], 1)
    out = _mlp_ln([x_dst, agg0, agg1],
                  [nw1[:_HID], nw1[_HID:], nw1[_HID:]], nb1, nw2, nb2,
                  ln=True, residuals=[x_dst] + list(extra_res),
                  bf16_copy=bf16_out)
    return out, msg


def _edge_emb(p, attr):
    w1, b1 = _w(p, 0)
    w2, b2 = _w(p, 1)
    return _mlp_ln([attr], [w1], b1, w2, b2, ln=True)


def _prep_edges(ei, attr, n_dst):
    e = ei.shape[1]
    ep = ((e + _EPAD - 1) // _EPAD) * _EPAD
    pad = ep - e
    src0 = jnp.pad(ei[0], (0, pad)).reshape(-1, _CHUNK)
    dst0 = jnp.pad(ei[1], (0, pad)).reshape(-1, _CHUNK)
    dstN = jnp.pad(ei[1], (0, pad),
                   constant_values=n_dst).reshape(-1, _CHUNK)
    attr_p = jnp.pad(attr, ((0, pad), (0, 0)))
    return src0, dst0, dstN, attr_p


def kernel(x, params, era_latlons, h_latlons, e2h_edge_attr, h2h_edge_attr,
           h2e_edge_attr, e2h_edge_index, h2h_edge_index, h2e_edge_index):
    bs = x.shape[0]
    x2 = x.reshape(bs * _ERA, -1)

    w1, b1 = _w(params["era_emb"], 0)
    w2, b2 = _w(params["era_emb"], 1)
    d_in = x2.shape[1]
    x_era = _mlp_ln([x2, era_latlons], [w1[:d_in], w1[d_in:]], b1, w2, b2)
    w1, b1 = _w(params["h_emb"], 0)
    w2, b2 = _w(params["h_emb"], 1)
    x_h = _mlp_ln([h_latlons], [w1], b1, w2, b2)

    src0, dst0, dstN, attr_p = _prep_edges(e2h_edge_index, e2h_edge_attr, _H)
    e_fmap = _edge_emb(params["fmap"]["edge_emb"], attr_p)
    x_latent, _ = _gnn_layer(params["fmap"]["layers"][0], x_era, x_h, x_h,
                             src0, dst0, dstN, [e_fmap], bs * _H, 1)

    src0, dst0, dstN, attr_p = _prep_edges(h2h_edge_index, h2h_edge_attr, _H)
    e_proc = _edge_emb(params["proc"]["edge_emb"], attr_p)
    x1, msg1 = _gnn_layer(params["proc"]["layers"][0], x_latent, x_latent,
                          x_latent, src0, dst0, dstN, [e_proc], bs * _H, 1)
    x_proc, _ = _gnn_layer(params["proc"]["layers"][1], x1, x1, x1,
                           src0, dst0, dstN, [e_proc, msg1], bs * _H, 1,
                           extra_res=[x_latent])

    src0, dst0, dstN, attr_p = _prep_edges(h2e_edge_index, h2e_edge_attr,
                                           bs * _ERA)
    e_bmap = _edge_emb(params["bmap"]["edge_emb"], attr_p)
    x_out, _ = _gnn_layer(params["bmap"]["layers"][0], x_proc, x_era,
                          x_era, src0, dst0, dstN, [e_bmap], bs * _ERA, 4)

    w1, b1 = _w(params["extract"], 0)
    w2, b2 = _w(params["extract"], 1)
    out_ch = w2.shape[1]
    out = _mlp_ln([x_out], [w1], b1, w2, b2, ln=False,
                  residuals=[x2[:, :out_ch]])
    return out.reshape(bs, _ERA, out_ch)


# speedup vs baseline: 1.7827x; 1.1195x over previous
import functools

import jax
import jax.numpy as jnp
from jax import lax
from jax.experimental import pallas as pl
from jax.experimental.pallas import tpu as pltpu
from jax.experimental.pallas import tpu_sc as plsc

_ERA = 50000
_H = 10000
_HID = 128
_NC = 2
_NS = 16
_NW = _NC * _NS
_CHUNK = 128
_EPAD = _NW * _CHUNK


def _sc_mesh():
    return plsc.VectorSubcoreMesh(core_axis_name="c", subcore_axis_name="s",
                                  num_cores=_NC, num_subcores=_NS)


def _mlp_ln(inputs, w1s, b1, w2, b2, *, ln=True, residuals=(), block=2048,
            bf16_copy=False):
    n_rows = inputs[0].shape[0]
    dout = w2.shape[1]
    n_in = len(inputs)
    n_res = len(residuals)
    grid = ((n_rows + block - 1) // block,)

    def body(*refs):
        in_refs = refs[:n_in]
        w1_refs = refs[n_in:2 * n_in]
        b1_ref = refs[2 * n_in]
        w2_ref = refs[2 * n_in + 1]
        b2_ref = refs[2 * n_in + 2]
        res_refs = refs[2 * n_in + 3:2 * n_in + 3 + n_res]
        acc = None
        for xr, wr in zip(in_refs, w1_refs):
            xv = xr[...]
            if xv.dtype != jnp.float32:
                xv = xv.astype(jnp.float32)
            t = jnp.dot(xv, wr[...], preferred_element_type=jnp.float32)
            acc = t if acc is None else acc + t
        acc = acc + b1_ref[...]
        h = acc * jax.nn.sigmoid(acc)
        y = jnp.dot(h, w2_ref[...], preferred_element_type=jnp.float32)
        y = y + b2_ref[...]
        if ln:
            m = jnp.mean(y, axis=-1, keepdims=True)
            yc = y - m
            v = jnp.mean(yc * yc, axis=-1, keepdims=True)
            y = yc * lax.rsqrt(v + 1e-5)
        for rr in res_refs:
            y = y + rr[...]
        if bf16_copy:
            refs[-2][...] = y
            refs[-1][...] = y.astype(jnp.bfloat16)
        else:
            refs[-1][...] = y

    in_specs = [pl.BlockSpec((block, a.shape[1]), lambda i: (i, 0))
                for a in inputs]
    in_specs += [pl.BlockSpec(w.shape, lambda i: (0, 0)) for w in w1s]
    b1_2d = b1.reshape(1, -1)
    b2_2d = b2.reshape(1, -1)
    in_specs += [pl.BlockSpec(b1_2d.shape, lambda i: (0, 0)),
                 pl.BlockSpec(w2.shape, lambda i: (0, 0)),
                 pl.BlockSpec(b2_2d.shape, lambda i: (0, 0))]
    in_specs += [pl.BlockSpec((block, r.shape[1]), lambda i: (i, 0))
                 for r in residuals]
    out_spec = pl.BlockSpec((block, dout), lambda i: (i, 0))
    if bf16_copy:
        out_specs = [out_spec, out_spec]
        out_shape = [jax.ShapeDtypeStruct((n_rows, dout), jnp.float32),
                     jax.ShapeDtypeStruct((n_rows, dout), jnp.bfloat16)]
    else:
        out_specs = out_spec
        out_shape = jax.ShapeDtypeStruct((n_rows, dout), jnp.float32)
    return pl.pallas_call(
        body,
        grid=grid,
        in_specs=in_specs,
        out_specs=out_specs,
        out_shape=out_shape,
    )(*inputs, *w1s, b1_2d, w2, b2_2d, *residuals)


def _sc_gather2(table_a, idx_a2, table_b, idx_b2):
    nchr = idx_a2.shape[0]
    ep = nchr * _CHUNK
    d = table_a.shape[1]
    dt = table_a.dtype
    nch = nchr // _NW
    grp = 3
    ngrp, tail = divmod(nch, grp)
    mesh = _sc_mesh()

    @functools.partial(
        pl.kernel,
        out_type=(jax.ShapeDtypeStruct((ep, d), dt),
                  jax.ShapeDtypeStruct((ep, d), dt)),
        mesh=mesh,
        scratch_types=[pltpu.VMEM((grp, _CHUNK), jnp.int32),
                       pltpu.VMEM((grp, _CHUNK), jnp.int32),
                       pltpu.VMEM((grp * _CHUNK, 128), dt),
                       pltpu.VMEM((grp * _CHUNK, 128), dt),
                       pltpu.SemaphoreType.DMA,
                       pltpu.SemaphoreType.DMA],
        compiler_params=pltpu.CompilerParams(use_tc_tiling_on_sc=False),
    )
    def k(ta, ia2, tb, ib2, oa, ob, iva2, ivb2, rva, rvb, sga, sgb):
        cid = lax.axis_index("c")
        sid = lax.axis_index("s")
        wid = sid * _NC + cid
        crow0 = wid * nch

        def do_group(cr, n):
            er = cr * _CHUNK
            pltpu.sync_copy(ia2.at[pl.ds(cr, n)], iva2.at[pl.ds(0, n)])
            pltpu.sync_copy(ib2.at[pl.ds(cr, n)], ivb2.at[pl.ds(0, n)])
            for q in range(n):
                pltpu.async_copy(ta.at[iva2.at[q]],
                                 rva.at[pl.ds(q * _CHUNK, _CHUNK)], sga)
                pltpu.async_copy(tb.at[ivb2.at[q]],
                                 rvb.at[pl.ds(q * _CHUNK, _CHUNK)], sgb)
            for q in range(n):
                pltpu.make_async_copy(
                    ta.at[iva2.at[q]],
                    rva.at[pl.ds(q * _CHUNK, _CHUNK)], sga).wait()
                pltpu.make_async_copy(
                    tb.at[ivb2.at[q]],
                    rvb.at[pl.ds(q * _CHUNK, _CHUNK)], sgb).wait()
            pltpu.sync_copy(rva.at[pl.ds(0, n * _CHUNK)],
                            oa.at[pl.ds(er, n * _CHUNK)])
            pltpu.sync_copy(rvb.at[pl.ds(0, n * _CHUNK)],
                            ob.at[pl.ds(er, n * _CHUNK)])

        def body(g, carry):
            do_group(crow0 + g * grp, grp)
            return carry

        lax.fori_loop(0, ngrp, body, 0)
        if tail:
            do_group(crow0 + ngrp * grp, tail)

    return k(table_a, idx_a2, table_b, idx_b2)


def _sc_segsum(msg, dst_pad, n_dst, n_slabs):
    ep, d = msg.shape
    sw = d // n_slabs
    npad = ((n_dst + 1 + 127) // 128) * 128
    bpw = ep // _NW
    nch = bpw // _CHUNK
    rows_pt = npad // _NS
    zrows = min(1024, rows_pt)
    zeros = jnp.zeros((zrows, sw), jnp.float32)
    mesh = _sc_mesh()

    grp = 2 if sw == d else 4
    ngrp, tail = divmod(nch, grp)
    nchr = ep // _CHUNK

    @functools.partial(
        pl.kernel,
        out_type=(jax.ShapeDtypeStruct((npad, d), jnp.float32),
                  jax.ShapeDtypeStruct((npad, d), jnp.float32)),
        mesh=mesh,
        scratch_types=[pltpu.VMEM((grp, _CHUNK), jnp.int32),
                       pltpu.VMEM((grp * _CHUNK, sw), jnp.float32),
                       pltpu.SemaphoreType.DMA,
                       pltpu.VMEM_SHARED((npad, sw), jnp.float32)],
        compiler_params=pltpu.CompilerParams(use_tc_tiling_on_sc=False),
    )
    def k(mh, ih2, zh, o0, o1, iv2, rv, sa, agg):
        cid = lax.axis_index("c")
        sid = lax.axis_index("s")
        wid = sid * _NC + cid
        crow0 = wid * nch
        tile_r0 = sid * rows_pt
        for slab in range(n_slabs):
            c0 = slab * sw
            r = 0
            while r < rows_pt:
                n = min(zrows, rows_pt - r)
                pltpu.sync_copy(zh.at[pl.ds(0, n)],
                                agg.at[pl.ds(tile_r0 + r, n)])
                r += n
            plsc.subcore_barrier()

            def do_group(cr, n):
                er = cr * _CHUNK
                pltpu.sync_copy(ih2.at[pl.ds(cr, n)], iv2.at[pl.ds(0, n)])
                pltpu.sync_copy(mh.at[pl.ds(er, n * _CHUNK), pl.ds(c0, sw)],
                                rv.at[pl.ds(0, n * _CHUNK)])
                for q in range(n):
                    pltpu.async_copy(rv.at[pl.ds(q * _CHUNK, _CHUNK)],
                                     agg.at[iv2.at[q]], sa, add=True)
                for q in range(n):
                    pltpu.make_async_copy(
                        rv.at[pl.ds(q * _CHUNK, _CHUNK)],
                        agg.at[iv2.at[q]], sa).wait()

            def body(g, carry):
                do_group(crow0 + g * grp, grp)
                return carry

            lax.fori_loop(0, ngrp, body, 0)
            if tail:
                do_group(crow0 + ngrp * grp, tail)
            plsc.subcore_barrier()

            @pl.when(cid == 0)
            def _():
                pltpu.sync_copy(agg.at[pl.ds(tile_r0, rows_pt)],
                                o0.at[pl.ds(tile_r0, rows_pt), pl.ds(c0, sw)])

            @pl.when(cid == 1)
            def _():
                pltpu.sync_copy(agg.at[pl.ds(tile_r0, rows_pt)],
                                o1.at[pl.ds(tile_r0, rows_pt), pl.ds(c0, sw)])

            plsc.subcore_barrier()

    return k(msg, dst_pad, zeros)


def _w(ps, i):
    return ps[i]["W"], ps[i]["b"]


def _gnn_layer(lp, src_tab16, dst_tab16, x_dst, src0, dst0, dstN, e_parts,
               n_dst, n_slabs, extra_res=(), bf16_out=False):
    src_g, dst_g = _sc_gather2(src_tab16, src0, dst_tab16, dst0)
    w1, b1 = _w(lp["msg"], 0)
    w2, b2 = _w(lp["msg"], 1)
    ins = [src_g, dst_g] + list(e_parts)
    w1s = [w1[:_HID], w1[_HID:2 * _HID]] + [w1[2 * _HID:]] * len(e_parts)
    msg = _mlp_ln(ins, w1s, b1, w2, b2, ln=True)
    agg0, agg1 = _sc_segsum(msg, dstN, n_dst, n_slabs)
    nw1, nb1 = _w(lp["node"], 0)
    nw2, nb2 = _w(lp["node"], 1)
    out = _mlp_ln([x_dst, agg0, agg1],
                  [nw1[:_HID], nw1[_HID:], nw1[_HID:]], nb1, nw2, nb2,
                  ln=True, residuals=[x_dst] + list(extra_res),
                  bf16_copy=bf16_out)
    return out, msg


def _edge_emb(p, attr):
    w1, b1 = _w(p, 0)
    w2, b2 = _w(p, 1)
    return _mlp_ln([attr], [w1], b1, w2, b2, ln=True)


def _prep_edges(ei, attr, n_dst):
    e = ei.shape[1]
    ep = ((e + _EPAD - 1) // _EPAD) * _EPAD
    pad = ep - e
    src0 = jnp.pad(ei[0], (0, pad)).reshape(-1, _CHUNK)
    dst0 = jnp.pad(ei[1], (0, pad)).reshape(-1, _CHUNK)
    dstN = jnp.pad(ei[1], (0, pad),
                   constant_values=n_dst).reshape(-1, _CHUNK)
    attr_p = jnp.pad(attr, ((0, pad), (0, 0)))
    return src0, dst0, dstN, attr_p


def kernel(x, params, era_latlons, h_latlons, e2h_edge_attr, h2h_edge_attr,
           h2e_edge_attr, e2h_edge_index, h2h_edge_index, h2e_edge_index):
    bs = x.shape[0]
    x2 = x.reshape(bs * _ERA, -1)

    w1, b1 = _w(params["era_emb"], 0)
    w2, b2 = _w(params["era_emb"], 1)
    d_in = x2.shape[1]
    x_era = _mlp_ln([x2, era_latlons], [w1[:d_in], w1[d_in:]], b1, w2, b2)
    w1, b1 = _w(params["h_emb"], 0)
    w2, b2 = _w(params["h_emb"], 1)
    x_h = _mlp_ln([h_latlons], [w1], b1, w2, b2)

    src0, dst0, dstN, attr_p = _prep_edges(e2h_edge_index, e2h_edge_attr, _H)
    e_fmap = _edge_emb(params["fmap"]["edge_emb"], attr_p)
    x_latent, _ = _gnn_layer(params["fmap"]["layers"][0], x_era, x_h, x_h,
                             src0, dst0, dstN, [e_fmap], bs * _H, 1)

    src0, dst0, dstN, attr_p = _prep_edges(h2h_edge_index, h2h_edge_attr, _H)
    e_proc = _edge_emb(params["proc"]["edge_emb"], attr_p)
    x1, msg1 = _gnn_layer(params["proc"]["layers"][0], x_latent, x_latent,
                          x_latent, src0, dst0, dstN, [e_proc], bs * _H, 1)
    x_proc, _ = _gnn_layer(params["proc"]["layers"][1], x1, x1, x1,
                           src0, dst0, dstN, [e_proc, msg1], bs * _H, 1,
                           extra_res=[x_latent])

    src0, dst0, dstN, attr_p = _prep_edges(h2e_edge_index, h2e_edge_attr,
                                           bs * _ERA)
    e_bmap = _edge_emb(params["bmap"]["edge_emb"], attr_p)
    x_out, _ = _gnn_layer(params["bmap"]["layers"][0], x_proc, x_era,
                          x_era, src0, dst0, dstN, [e_bmap], bs * _ERA, 4)

    w1, b1 = _w(params["extract"], 0)
    w2, b2 = _w(params["extract"], 1)
    out_ch = w2.shape[1]
    out = _mlp_ln([x_out], [w1], b1, w2, b2, ln=False,
                  residuals=[x2[:, :out_ch]])
    return out.reshape(bs, _ERA, out_ch)


# speedup vs baseline: 1.8910x; 1.0607x over previous
import functools

import jax
import jax.numpy as jnp
from jax import lax
from jax.experimental import pallas as pl
from jax.experimental.pallas import tpu as pltpu
from jax.experimental.pallas import tpu_sc as plsc

_ERA = 50000
_H = 10000
_HID = 128
_NC = 2
_NS = 16
_NW = _NC * _NS
_CHUNK = 128
_EPAD = _NW * _CHUNK


def _sc_mesh():
    return plsc.VectorSubcoreMesh(core_axis_name="c", subcore_axis_name="s",
                                  num_cores=_NC, num_subcores=_NS)


def _mlp_ln(inputs, w1s, b1, w2, b2, *, ln=True, residuals=(), block=4096,
            bf16_copy=False):
    n_rows = inputs[0].shape[0]
    dout = w2.shape[1]
    n_in = len(inputs)
    n_res = len(residuals)
    grid = ((n_rows + block - 1) // block,)

    def body(*refs):
        in_refs = refs[:n_in]
        w1_refs = refs[n_in:2 * n_in]
        b1_ref = refs[2 * n_in]
        w2_ref = refs[2 * n_in + 1]
        b2_ref = refs[2 * n_in + 2]
        res_refs = refs[2 * n_in + 3:2 * n_in + 3 + n_res]
        acc = None
        for xr, wr in zip(in_refs, w1_refs):
            xv = xr[...]
            if xv.dtype != jnp.float32:
                xv = xv.astype(jnp.float32)
            t = jnp.dot(xv, wr[...], preferred_element_type=jnp.float32)
            acc = t if acc is None else acc + t
        acc = acc + b1_ref[...]
        h = acc * jax.nn.sigmoid(acc)
        y = jnp.dot(h, w2_ref[...], preferred_element_type=jnp.float32)
        y = y + b2_ref[...]
        if ln:
            m = jnp.mean(y, axis=-1, keepdims=True)
            yc = y - m
            v = jnp.mean(yc * yc, axis=-1, keepdims=True)
            y = yc * lax.rsqrt(v + 1e-5)
        for rr in res_refs:
            y = y + rr[...]
        if bf16_copy:
            refs[-2][...] = y
            refs[-1][...] = y.astype(jnp.bfloat16)
        else:
            refs[-1][...] = y

    in_specs = [pl.BlockSpec((block, a.shape[1]), lambda i: (i, 0))
                for a in inputs]
    in_specs += [pl.BlockSpec(w.shape, lambda i: (0, 0)) for w in w1s]
    b1_2d = b1.reshape(1, -1)
    b2_2d = b2.reshape(1, -1)
    in_specs += [pl.BlockSpec(b1_2d.shape, lambda i: (0, 0)),
                 pl.BlockSpec(w2.shape, lambda i: (0, 0)),
                 pl.BlockSpec(b2_2d.shape, lambda i: (0, 0))]
    in_specs += [pl.BlockSpec((block, r.shape[1]), lambda i: (i, 0))
                 for r in residuals]
    out_spec = pl.BlockSpec((block, dout), lambda i: (i, 0))
    if bf16_copy:
        out_specs = [out_spec, out_spec]
        out_shape = [jax.ShapeDtypeStruct((n_rows, dout), jnp.float32),
                     jax.ShapeDtypeStruct((n_rows, dout), jnp.bfloat16)]
    else:
        out_specs = out_spec
        out_shape = jax.ShapeDtypeStruct((n_rows, dout), jnp.float32)
    return pl.pallas_call(
        body,
        grid=grid,
        in_specs=in_specs,
        out_specs=out_specs,
        out_shape=out_shape,
    )(*inputs, *w1s, b1_2d, w2, b2_2d, *residuals)


def _sc_gather2(table_a, idx_a2, table_b, idx_b2):
    nchr = idx_a2.shape[0]
    ep = nchr * _CHUNK
    d = table_a.shape[1]
    dt = table_a.dtype
    nch = nchr // _NW
    grp = 3
    ngrp, tail = divmod(nch, grp)
    mesh = _sc_mesh()

    @functools.partial(
        pl.kernel,
        out_type=(jax.ShapeDtypeStruct((ep, d), dt),
                  jax.ShapeDtypeStruct((ep, d), dt)),
        mesh=mesh,
        scratch_types=[pltpu.VMEM((grp, _CHUNK), jnp.int32),
                       pltpu.VMEM((grp, _CHUNK), jnp.int32),
                       pltpu.VMEM((grp * _CHUNK, 128), dt),
                       pltpu.VMEM((grp * _CHUNK, 128), dt),
                       pltpu.SemaphoreType.DMA,
                       pltpu.SemaphoreType.DMA],
        compiler_params=pltpu.CompilerParams(use_tc_tiling_on_sc=False),
    )
    def k(ta, ia2, tb, ib2, oa, ob, iva2, ivb2, rva, rvb, sga, sgb):
        cid = lax.axis_index("c")
        sid = lax.axis_index("s")
        wid = sid * _NC + cid
        crow0 = wid * nch

        def do_group(cr, n):
            er = cr * _CHUNK
            pltpu.sync_copy(ia2.at[pl.ds(cr, n)], iva2.at[pl.ds(0, n)])
            pltpu.sync_copy(ib2.at[pl.ds(cr, n)], ivb2.at[pl.ds(0, n)])
            for q in range(n):
                pltpu.async_copy(ta.at[iva2.at[q]],
                                 rva.at[pl.ds(q * _CHUNK, _CHUNK)], sga)
                pltpu.async_copy(tb.at[ivb2.at[q]],
                                 rvb.at[pl.ds(q * _CHUNK, _CHUNK)], sgb)
            for q in range(n):
                pltpu.make_async_copy(
                    ta.at[iva2.at[q]],
                    rva.at[pl.ds(q * _CHUNK, _CHUNK)], sga).wait()
                pltpu.make_async_copy(
                    tb.at[ivb2.at[q]],
                    rvb.at[pl.ds(q * _CHUNK, _CHUNK)], sgb).wait()
            pltpu.sync_copy(rva.at[pl.ds(0, n * _CHUNK)],
                            oa.at[pl.ds(er, n * _CHUNK)])
            pltpu.sync_copy(rvb.at[pl.ds(0, n * _CHUNK)],
                            ob.at[pl.ds(er, n * _CHUNK)])

        def body(g, carry):
            do_group(crow0 + g * grp, grp)
            return carry

        lax.fori_loop(0, ngrp, body, 0)
        if tail:
            do_group(crow0 + ngrp * grp, tail)

    return k(table_a, idx_a2, table_b, idx_b2)


def _sc_segsum(msg, dst_pad, n_dst, n_slabs):
    ep, d = msg.shape
    sw = d // n_slabs
    npad = ((n_dst + 1 + 127) // 128) * 128
    bpw = ep // _NW
    nch = bpw // _CHUNK
    rows_pt = npad // _NS
    zrows = min(1024, rows_pt)
    zeros = jnp.zeros((zrows, sw), jnp.float32)
    mesh = _sc_mesh()

    grp = 2 if sw == d else 4
    ngrp, tail = divmod(nch, grp)
    nchr = ep // _CHUNK

    @functools.partial(
        pl.kernel,
        out_type=(jax.ShapeDtypeStruct((npad, d), jnp.float32),
                  jax.ShapeDtypeStruct((npad, d), jnp.float32)),
        mesh=mesh,
        scratch_types=[pltpu.VMEM((grp, _CHUNK), jnp.int32),
                       pltpu.VMEM((grp * _CHUNK, sw), jnp.float32),
                       pltpu.SemaphoreType.DMA,
                       pltpu.VMEM_SHARED((npad, sw), jnp.float32)],
        compiler_params=pltpu.CompilerParams(use_tc_tiling_on_sc=False),
    )
    def k(mh, ih2, zh, o0, o1, iv2, rv, sa, agg):
        cid = lax.axis_index("c")
        sid = lax.axis_index("s")
        wid = sid * _NC + cid
        crow0 = wid * nch
        tile_r0 = sid * rows_pt
        for slab in range(n_slabs):
            c0 = slab * sw
            r = 0
            while r < rows_pt:
                n = min(zrows, rows_pt - r)
                pltpu.sync_copy(zh.at[pl.ds(0, n)],
                                agg.at[pl.ds(tile_r0 + r, n)])
                r += n
            plsc.subcore_barrier()

            def do_group(cr, n):
                er = cr * _CHUNK
                pltpu.sync_copy(ih2.at[pl.ds(cr, n)], iv2.at[pl.ds(0, n)])
                pltpu.sync_copy(mh.at[pl.ds(er, n * _CHUNK), pl.ds(c0, sw)],
                                rv.at[pl.ds(0, n * _CHUNK)])
                for q in range(n):
                    pltpu.async_copy(rv.at[pl.ds(q * _CHUNK, _CHUNK)],
                                     agg.at[iv2.at[q]], sa, add=True)
                for q in range(n):
                    pltpu.make_async_copy(
                        rv.at[pl.ds(q * _CHUNK, _CHUNK)],
                        agg.at[iv2.at[q]], sa).wait()

            def body(g, carry):
                do_group(crow0 + g * grp, grp)
                return carry

            lax.fori_loop(0, ngrp, body, 0)
            if tail:
                do_group(crow0 + ngrp * grp, tail)
            plsc.subcore_barrier()

            @pl.when(cid == 0)
            def _():
                pltpu.sync_copy(agg.at[pl.ds(tile_r0, rows_pt)],
                                o0.at[pl.ds(tile_r0, rows_pt), pl.ds(c0, sw)])

            @pl.when(cid == 1)
            def _():
                pltpu.sync_copy(agg.at[pl.ds(tile_r0, rows_pt)],
                                o1.at[pl.ds(tile_r0, rows_pt), pl.ds(c0, sw)])

            plsc.subcore_barrier()

    return k(msg, dst_pad, zeros)


def _w(ps, i):
    return ps[i]["W"], ps[i]["b"]


def _gnn_layer(lp, src_tab16, dst_tab16, x_dst, src0, dst0, dstN, e_parts,
               n_dst, n_slabs, extra_res=(), bf16_out=False):
    src_g, dst_g = _sc_gather2(src_tab16, src0, dst_tab16, dst0)
    w1, b1 = _w(lp["msg"], 0)
    w2, b2 = _w(lp["msg"], 1)
    ins = [src_g, dst_g] + list(e_parts)
    w1s = [w1[:_HID], w1[_HID:2 * _HID]] + [w1[2 * _HID:]] * len(e_parts)
    msg = _mlp_ln(ins, w1s, b1, w2, b2, ln=True)
    agg0, agg1 = _sc_segsum(msg, dstN, n_dst, n_slabs)
    nw1, nb1 = _w(lp["node"], 0)
    nw2, nb2 = _w(lp["node"], 1)
    out = _mlp_ln([x_dst, agg0, agg1],
                  [nw1[:_HID], nw1[_HID:], nw1[_HID:]], nb1, nw2, nb2,
                  ln=True, residuals=[x_dst] + list(extra_res),
                  bf16_copy=bf16_out)
    return out, msg


def _edge_emb(p, attr):
    w1, b1 = _w(p, 0)
    w2, b2 = _w(p, 1)
    return _mlp_ln([attr], [w1], b1, w2, b2, ln=True)


def _prep_edges(ei, attr, n_dst):
    e = ei.shape[1]
    ep = ((e + _EPAD - 1) // _EPAD) * _EPAD
    pad = ep - e
    src0 = jnp.pad(ei[0], (0, pad)).reshape(-1, _CHUNK)
    dst0 = jnp.pad(ei[1], (0, pad)).reshape(-1, _CHUNK)
    dstN = jnp.pad(ei[1], (0, pad),
                   constant_values=n_dst).reshape(-1, _CHUNK)
    attr_p = jnp.pad(attr, ((0, pad), (0, 0)))
    return src0, dst0, dstN, attr_p


def kernel(x, params, era_latlons, h_latlons, e2h_edge_attr, h2h_edge_attr,
           h2e_edge_attr, e2h_edge_index, h2h_edge_index, h2e_edge_index):
    bs = x.shape[0]
    x2 = x.reshape(bs * _ERA, -1)

    w1, b1 = _w(params["era_emb"], 0)
    w2, b2 = _w(params["era_emb"], 1)
    d_in = x2.shape[1]
    x_era = _mlp_ln([x2, era_latlons], [w1[:d_in], w1[d_in:]], b1, w2, b2)
    w1, b1 = _w(params["h_emb"], 0)
    w2, b2 = _w(params["h_emb"], 1)
    x_h = _mlp_ln([h_latlons], [w1], b1, w2, b2)

    src0, dst0, dstN, attr_p = _prep_edges(e2h_edge_index, e2h_edge_attr, _H)
    e_fmap = _edge_emb(params["fmap"]["edge_emb"], attr_p)
    x_latent, _ = _gnn_layer(params["fmap"]["layers"][0], x_era, x_h, x_h,
                             src0, dst0, dstN, [e_fmap], bs * _H, 1)

    src0, dst0, dstN, attr_p = _prep_edges(h2h_edge_index, h2h_edge_attr, _H)
    e_proc = _edge_emb(params["proc"]["edge_emb"], attr_p)
    x1, msg1 = _gnn_layer(params["proc"]["layers"][0], x_latent, x_latent,
                          x_latent, src0, dst0, dstN, [e_proc], bs * _H, 1)
    x_proc, _ = _gnn_layer(params["proc"]["layers"][1], x1, x1, x1,
                           src0, dst0, dstN, [e_proc, msg1], bs * _H, 1,
                           extra_res=[x_latent])

    src0, dst0, dstN, attr_p = _prep_edges(h2e_edge_index, h2e_edge_attr,
                                           bs * _ERA)
    e_bmap = _edge_emb(params["bmap"]["edge_emb"], attr_p)
    x_out, _ = _gnn_layer(params["bmap"]["layers"][0], x_proc, x_era,
                          x_era, src0, dst0, dstN, [e_bmap], bs * _ERA, 4)

    w1, b1 = _w(params["extract"], 0)
    w2, b2 = _w(params["extract"], 1)
    out_ch = w2.shape[1]
    out = _mlp_ln([x_out], [w1], b1, w2, b2, ln=False,
                  residuals=[x2[:, :out_ch]])
    return out.reshape(bs, _ERA, out_ch)


# speedup vs baseline: 1.9109x; 1.0105x over previous
import functools

import jax
import jax.numpy as jnp
from jax import lax
from jax.experimental import pallas as pl
from jax.experimental.pallas import tpu as pltpu
from jax.experimental.pallas import tpu_sc as plsc

_ERA = 50000
_H = 10000
_HID = 128
_NC = 2
_NS = 16
_NW = _NC * _NS
_CHUNK = 128
_EPAD = _NW * _CHUNK


def _sc_mesh():
    return plsc.VectorSubcoreMesh(core_axis_name="c", subcore_axis_name="s",
                                  num_cores=_NC, num_subcores=_NS)


def _mlp_ln(inputs, w1s, b1, w2, b2, *, ln=True, residuals=(), block=8192,
            bf16_copy=False):
    n_rows = inputs[0].shape[0]
    dout = w2.shape[1]
    n_in = len(inputs)
    n_res = len(residuals)
    grid = ((n_rows + block - 1) // block,)

    def body(*refs):
        in_refs = refs[:n_in]
        w1_refs = refs[n_in:2 * n_in]
        b1_ref = refs[2 * n_in]
        w2_ref = refs[2 * n_in + 1]
        b2_ref = refs[2 * n_in + 2]
        res_refs = refs[2 * n_in + 3:2 * n_in + 3 + n_res]
        acc = None
        for xr, wr in zip(in_refs, w1_refs):
            xv = xr[...]
            if xv.dtype != jnp.float32:
                xv = xv.astype(jnp.float32)
            t = jnp.dot(xv, wr[...], preferred_element_type=jnp.float32)
            acc = t if acc is None else acc + t
        acc = acc + b1_ref[...]
        h = acc * jax.nn.sigmoid(acc)
        y = jnp.dot(h, w2_ref[...], preferred_element_type=jnp.float32)
        y = y + b2_ref[...]
        if ln:
            m = jnp.mean(y, axis=-1, keepdims=True)
            yc = y - m
            v = jnp.mean(yc * yc, axis=-1, keepdims=True)
            y = yc * lax.rsqrt(v + 1e-5)
        for rr in res_refs:
            y = y + rr[...]
        if bf16_copy:
            refs[-2][...] = y
            refs[-1][...] = y.astype(jnp.bfloat16)
        else:
            refs[-1][...] = y

    in_specs = [pl.BlockSpec((block, a.shape[1]), lambda i: (i, 0))
                for a in inputs]
    in_specs += [pl.BlockSpec(w.shape, lambda i: (0, 0)) for w in w1s]
    b1_2d = b1.reshape(1, -1)
    b2_2d = b2.reshape(1, -1)
    in_specs += [pl.BlockSpec(b1_2d.shape, lambda i: (0, 0)),
                 pl.BlockSpec(w2.shape, lambda i: (0, 0)),
                 pl.BlockSpec(b2_2d.shape, lambda i: (0, 0))]
    in_specs += [pl.BlockSpec((block, r.shape[1]), lambda i: (i, 0))
                 for r in residuals]
    out_spec = pl.BlockSpec((block, dout), lambda i: (i, 0))
    if bf16_copy:
        out_specs = [out_spec, out_spec]
        out_shape = [jax.ShapeDtypeStruct((n_rows, dout), jnp.float32),
                     jax.ShapeDtypeStruct((n_rows, dout), jnp.bfloat16)]
    else:
        out_specs = out_spec
        out_shape = jax.ShapeDtypeStruct((n_rows, dout), jnp.float32)
    return pl.pallas_call(
        body,
        grid=grid,
        in_specs=in_specs,
        out_specs=out_specs,
        out_shape=out_shape,
    )(*inputs, *w1s, b1_2d, w2, b2_2d, *residuals)


def _sc_gather2(table_a, idx_a2, table_b, idx_b2):
    nchr = idx_a2.shape[0]
    ep = nchr * _CHUNK
    d = table_a.shape[1]
    dt = table_a.dtype
    nch = nchr // _NW
    grp = 3
    ngrp, tail = divmod(nch, grp)
    mesh = _sc_mesh()

    @functools.partial(
        pl.kernel,
        out_type=(jax.ShapeDtypeStruct((ep, d), dt),
                  jax.ShapeDtypeStruct((ep, d), dt)),
        mesh=mesh,
        scratch_types=[pltpu.VMEM((grp, _CHUNK), jnp.int32),
                       pltpu.VMEM((grp, _CHUNK), jnp.int32),
                       pltpu.VMEM((grp * _CHUNK, 128), dt),
                       pltpu.VMEM((grp * _CHUNK, 128), dt),
                       pltpu.SemaphoreType.DMA,
                       pltpu.SemaphoreType.DMA],
        compiler_params=pltpu.CompilerParams(use_tc_tiling_on_sc=False),
    )
    def k(ta, ia2, tb, ib2, oa, ob, iva2, ivb2, rva, rvb, sga, sgb):
        cid = lax.axis_index("c")
        sid = lax.axis_index("s")
        wid = sid * _NC + cid
        crow0 = wid * nch

        def do_group(cr, n):
            er = cr * _CHUNK
            pltpu.sync_copy(ia2.at[pl.ds(cr, n)], iva2.at[pl.ds(0, n)])
            pltpu.sync_copy(ib2.at[pl.ds(cr, n)], ivb2.at[pl.ds(0, n)])
            for q in range(n):
                pltpu.async_copy(ta.at[iva2.at[q]],
                                 rva.at[pl.ds(q * _CHUNK, _CHUNK)], sga)
                pltpu.async_copy(tb.at[ivb2.at[q]],
                                 rvb.at[pl.ds(q * _CHUNK, _CHUNK)], sgb)
            for q in range(n):
                pltpu.make_async_copy(
                    ta.at[iva2.at[q]],
                    rva.at[pl.ds(q * _CHUNK, _CHUNK)], sga).wait()
                pltpu.make_async_copy(
                    tb.at[ivb2.at[q]],
                    rvb.at[pl.ds(q * _CHUNK, _CHUNK)], sgb).wait()
            pltpu.sync_copy(rva.at[pl.ds(0, n * _CHUNK)],
                            oa.at[pl.ds(er, n * _CHUNK)])
            pltpu.sync_copy(rvb.at[pl.ds(0, n * _CHUNK)],
                            ob.at[pl.ds(er, n * _CHUNK)])

        def body(g, carry):
            do_group(crow0 + g * grp, grp)
            return carry

        lax.fori_loop(0, ngrp, body, 0)
        if tail:
            do_group(crow0 + ngrp * grp, tail)

    return k(table_a, idx_a2, table_b, idx_b2)


def _sc_segsum(msg, dst_pad, n_dst, n_slabs):
    ep, d = msg.shape
    sw = d // n_slabs
    npad = ((n_dst + 1 + 127) // 128) * 128
    bpw = ep // _NW
    nch = bpw // _CHUNK
    rows_pt = npad // _NS
    zrows = min(1024, rows_pt)
    zeros = jnp.zeros((zrows, sw), jnp.float32)
    mesh = _sc_mesh()

    grp = 2 if sw == d else 4
    ngrp, tail = divmod(nch, grp)
    nchr = ep // _CHUNK

    @functools.partial(
        pl.kernel,
        out_type=(jax.ShapeDtypeStruct((npad, d), jnp.float32),
                  jax.ShapeDtypeStruct((npad, d), jnp.float32)),
        mesh=mesh,
        scratch_types=[pltpu.VMEM((grp, _CHUNK), jnp.int32),
                       pltpu.VMEM((grp * _CHUNK, sw), jnp.float32),
                       pltpu.SemaphoreType.DMA,
                       pltpu.VMEM_SHARED((npad, sw), jnp.float32)],
        compiler_params=pltpu.CompilerParams(use_tc_tiling_on_sc=False),
    )
    def k(mh, ih2, zh, o0, o1, iv2, rv, sa, agg):
        cid = lax.axis_index("c")
        sid = lax.axis_index("s")
        wid = sid * _NC + cid
        crow0 = wid * nch
        tile_r0 = sid * rows_pt
        for slab in range(n_slabs):
            c0 = slab * sw
            r = 0
            while r < rows_pt:
                n = min(zrows, rows_pt - r)
                pltpu.sync_copy(zh.at[pl.ds(0, n)],
                                agg.at[pl.ds(tile_r0 + r, n)])
                r += n
            plsc.subcore_barrier()

            def do_group(cr, n):
                er = cr * _CHUNK
                pltpu.sync_copy(ih2.at[pl.ds(cr, n)], iv2.at[pl.ds(0, n)])
                pltpu.sync_copy(mh.at[pl.ds(er, n * _CHUNK), pl.ds(c0, sw)],
                                rv.at[pl.ds(0, n * _CHUNK)])
                for q in range(n):
                    pltpu.async_copy(rv.at[pl.ds(q * _CHUNK, _CHUNK)],
                                     agg.at[iv2.at[q]], sa, add=True)
                for q in range(n):
                    pltpu.make_async_copy(
                        rv.at[pl.ds(q * _CHUNK, _CHUNK)],
                        agg.at[iv2.at[q]], sa).wait()

            def body(g, carry):
                do_group(crow0 + g * grp, grp)
                return carry

            lax.fori_loop(0, ngrp, body, 0)
            if tail:
                do_group(crow0 + ngrp * grp, tail)
            plsc.subcore_barrier()

            @pl.when(cid == 0)
            def _():
                pltpu.sync_copy(agg.at[pl.ds(tile_r0, rows_pt)],
                                o0.at[pl.ds(tile_r0, rows_pt), pl.ds(c0, sw)])

            @pl.when(cid == 1)
            def _():
                pltpu.sync_copy(agg.at[pl.ds(tile_r0, rows_pt)],
                                o1.at[pl.ds(tile_r0, rows_pt), pl.ds(c0, sw)])

            plsc.subcore_barrier()

    return k(msg, dst_pad, zeros)


def _w(ps, i):
    return ps[i]["W"], ps[i]["b"]


def _gnn_layer(lp, src_tab16, dst_tab16, x_dst, src0, dst0, dstN, e_parts,
               n_dst, n_slabs, extra_res=(), bf16_out=False):
    src_g, dst_g = _sc_gather2(src_tab16, src0, dst_tab16, dst0)
    w1, b1 = _w(lp["msg"], 0)
    w2, b2 = _w(lp["msg"], 1)
    ins = [src_g, dst_g] + list(e_parts)
    w1s = [w1[:_HID], w1[_HID:2 * _HID]] + [w1[2 * _HID:]] * len(e_parts)
    msg = _mlp_ln(ins, w1s, b1, w2, b2, ln=True)
    agg0, agg1 = _sc_segsum(msg, dstN, n_dst, n_slabs)
    nw1, nb1 = _w(lp["node"], 0)
    nw2, nb2 = _w(lp["node"], 1)
    out = _mlp_ln([x_dst, agg0, agg1],
                  [nw1[:_HID], nw1[_HID:], nw1[_HID:]], nb1, nw2, nb2,
                  ln=True, residuals=[x_dst] + list(extra_res),
                  bf16_copy=bf16_out)
    return out, msg


def _edge_emb(p, attr):
    w1, b1 = _w(p, 0)
    w2, b2 = _w(p, 1)
    return _mlp_ln([attr], [w1], b1, w2, b2, ln=True)


def _prep_edges(ei, attr, n_dst):
    e = ei.shape[1]
    ep = ((e + _EPAD - 1) // _EPAD) * _EPAD
    pad = ep - e
    src0 = jnp.pad(ei[0], (0, pad)).reshape(-1, _CHUNK)
    dst0 = jnp.pad(ei[1], (0, pad)).reshape(-1, _CHUNK)
    dstN = jnp.pad(ei[1], (0, pad),
                   constant_values=n_dst).reshape(-1, _CHUNK)
    attr_p = jnp.pad(attr, ((0, pad), (0, 0)))
    return src0, dst0, dstN, attr_p


def kernel(x, params, era_latlons, h_latlons, e2h_edge_attr, h2h_edge_attr,
           h2e_edge_attr, e2h_edge_index, h2h_edge_index, h2e_edge_index):
    bs = x.shape[0]
    x2 = x.reshape(bs * _ERA, -1)

    w1, b1 = _w(params["era_emb"], 0)
    w2, b2 = _w(params["era_emb"], 1)
    d_in = x2.shape[1]
    x_era = _mlp_ln([x2, era_latlons], [w1[:d_in], w1[d_in:]], b1, w2, b2)
    w1, b1 = _w(params["h_emb"], 0)
    w2, b2 = _w(params["h_emb"], 1)
    x_h = _mlp_ln([h_latlons], [w1], b1, w2, b2)

    src0, dst0, dstN, attr_p = _prep_edges(e2h_edge_index, e2h_edge_attr, _H)
    e_fmap = _edge_emb(params["fmap"]["edge_emb"], attr_p)
    x_latent, _ = _gnn_layer(params["fmap"]["layers"][0], x_era, x_h, x_h,
                             src0, dst0, dstN, [e_fmap], bs * _H, 1)

    src0, dst0, dstN, attr_p = _prep_edges(h2h_edge_index, h2h_edge_attr, _H)
    e_proc = _edge_emb(params["proc"]["edge_emb"], attr_p)
    x1, msg1 = _gnn_layer(params["proc"]["layers"][0], x_latent, x_latent,
                          x_latent, src0, dst0, dstN, [e_proc], bs * _H, 1)
    x_proc, _ = _gnn_layer(params["proc"]["layers"][1], x1, x1, x1,
                           src0, dst0, dstN, [e_proc, msg1], bs * _H, 1,
                           extra_res=[x_latent])

    src0, dst0, dstN, attr_p = _prep_edges(h2e_edge_index, h2e_edge_attr,
                                           bs * _ERA)
    e_bmap = _edge_emb(params["bmap"]["edge_emb"], attr_p)
    x_out, _ = _gnn_layer(params["bmap"]["layers"][0], x_proc, x_era,
                          x_era, src0, dst0, dstN, [e_bmap], bs * _ERA, 4)

    w1, b1 = _w(params["extract"], 0)
    w2, b2 = _w(params["extract"], 1)
    out_ch = w2.shape[1]
    out = _mlp_ln([x_out], [w1], b1, w2, b2, ln=False,
                  residuals=[x2[:, :out_ch]])
    return out.reshape(bs, _ERA, out_ch)
